# baseline (speedup 1.0000x reference)
import functools

import jax
import jax.numpy as jnp
from jax import lax
from jax.experimental import pallas as pl
from jax.experimental.pallas import tpu as pltpu
from jax.experimental.pallas import tpu_sc as plsc

BATCH = 4
SEQ = 8192
DIM = 1024

NC = 2
NS = 16
NW = NC * NS

ROWS_PER_W = SEQ // NW
CHUNK = 8
CE = CHUNK * DIM
NCHUNK = ROWS_PER_W // CHUNK
VECS = CE // 16
UNROLL = 8

_mesh = plsc.VectorSubcoreMesh(
    core_axis_name="c", subcore_axis_name="s", num_cores=NC, num_subcores=NS
)


def _add_chunk(dst, src):
    def body(i, _):
        base = i * (16 * UNROLL)
        for u in range(UNROLL):
            off = base + u * 16
            plsc.addupdate(dst.at[pl.ds(off, 16)], src[pl.ds(off, 16)])
        return 0

    lax.fori_loop(0, VECS // UNROLL, body, 0, unroll=False)


@functools.partial(
    pl.kernel,
    out_type=jax.ShapeDtypeStruct((BATCH * SEQ * DIM,), jnp.float32),
    mesh=_mesh,
    scratch_types=[
        pltpu.VMEM((8, CE), jnp.float32),
        pltpu.VMEM((2, CE), jnp.float32),
        pltpu.SemaphoreType.DMA((8,)),
        pltpu.SemaphoreType.DMA((8,)),
        pltpu.SemaphoreType.DMA((2,)),
    ],
)
def _sc_add(in_hbm, emb_hbm, out_hbm, io_v, emb_v, isems, osems, esems):
    wid = lax.axis_index("s") * NC + lax.axis_index("c")
    ebase = wid * (ROWS_PER_W * DIM)

    def emb_off(k):
        return ebase + k * CE

    def io_off(k, b):
        return b * (SEQ * DIM) + ebase + k * CE

    def issue_emb(k, slot):
        pltpu.async_copy(
            emb_hbm.at[pl.ds(emb_off(k), CE)], emb_v.at[slot], esems.at[slot]
        )

    def issue_in(k, b, slot):
        pltpu.async_copy(
            in_hbm.at[pl.ds(io_off(k, b), CE)], io_v.at[slot], isems.at[slot]
        )

    def issue_out(k, b, slot):
        pltpu.async_copy(
            io_v.at[slot], out_hbm.at[pl.ds(io_off(k, b), CE)], osems.at[slot]
        )

    def wait_in(slot):
        pltpu.make_async_copy(
            in_hbm.at[pl.ds(0, CE)], io_v.at[slot], isems.at[slot]
        ).wait()

    def wait_out(slot):
        pltpu.make_async_copy(
            io_v.at[slot], out_hbm.at[pl.ds(0, CE)], osems.at[slot]
        ).wait()

    def wait_emb(slot):
        pltpu.make_async_copy(
            emb_hbm.at[pl.ds(0, CE)], emb_v.at[slot], esems.at[slot]
        ).wait()

    issue_emb(0, 0)
    for b in range(BATCH):
        issue_in(0, b, b)

    def step(k, cur, nxt):
        ecur = cur // 4
        enxt = nxt // 4
        wait_emb(ecur)

        @pl.when(k + 1 < NCHUNK)
        def _():
            issue_emb(k + 1, enxt)

        for b in range(BATCH):
            wait_in(cur + b)

            @pl.when(k >= 1)
            def _():
                wait_out(nxt + b)

            @pl.when(k + 1 < NCHUNK)
            def _():
                issue_in(k + 1, b, nxt + b)

            _add_chunk(io_v.at[cur + b], emb_v.at[ecur])
            issue_out(k, b, cur + b)

    def two_steps(kk, _):
        step(2 * kk, 0, 4)
        step(2 * kk + 1, 4, 0)
        return 0

    lax.fori_loop(0, NCHUNK // 2, two_steps, 0, unroll=False)

    for b in range(BATCH):
        wait_out(4 + b)


def kernel(inputs, embeddings):
    seq_len = inputs.shape[1]
    pos = embeddings[:seq_len].reshape(-1)
    flat = inputs.reshape(-1)
    out = _sc_add(flat, pos)
    return out.reshape(inputs.shape)


# speedup vs baseline: 1.3440x; 1.3440x over previous
import functools

import jax
import jax.numpy as jnp
from jax import lax
from jax.experimental import pallas as pl
from jax.experimental.pallas import tpu as pltpu
from jax.experimental.pallas import tpu_sc as plsc

BATCH = 4
SEQ = 8192
DIM = 1024

NC = 2
NS = 16
NW = NC * NS

ROWS_PER_W = SEQ // NW
CHUNK = 8
CE = CHUNK * DIM
NCHUNK = ROWS_PER_W // CHUNK
VECS = CE // 16
UNROLL = 8

_mesh = plsc.VectorSubcoreMesh(
    core_axis_name="c", subcore_axis_name="s", num_cores=NC, num_subcores=NS
)


def _add_chunk(dst, src):
    @plsc.parallel_loop(0, CE, 16, unroll=UNROLL)
    def _(off):
        plsc.addupdate(dst.at[pl.ds(off, 16)], src[pl.ds(off, 16)])


@functools.partial(
    pl.kernel,
    out_type=jax.ShapeDtypeStruct((BATCH * SEQ * DIM,), jnp.float32),
    mesh=_mesh,
    scratch_types=[
        pltpu.VMEM((8, CE), jnp.float32),
        pltpu.VMEM((2, CE), jnp.float32),
        pltpu.SemaphoreType.DMA((8,)),
        pltpu.SemaphoreType.DMA((8,)),
        pltpu.SemaphoreType.DMA((2,)),
    ],
)
def _sc_add(in_hbm, emb_hbm, out_hbm, io_v, emb_v, isems, osems, esems):
    wid = lax.axis_index("s") * NC + lax.axis_index("c")
    ebase = wid * (ROWS_PER_W * DIM)

    def emb_off(k):
        return ebase + k * CE

    def io_off(k, b):
        return b * (SEQ * DIM) + ebase + k * CE

    def issue_emb(k, slot):
        pltpu.async_copy(
            emb_hbm.at[pl.ds(emb_off(k), CE)], emb_v.at[slot], esems.at[slot]
        )

    def issue_in(k, b, slot):
        pltpu.async_copy(
            in_hbm.at[pl.ds(io_off(k, b), CE)], io_v.at[slot], isems.at[slot]
        )

    def issue_out(k, b, slot):
        pltpu.async_copy(
            io_v.at[slot], out_hbm.at[pl.ds(io_off(k, b), CE)], osems.at[slot]
        )

    def wait_in(slot):
        pltpu.make_async_copy(
            in_hbm.at[pl.ds(0, CE)], io_v.at[slot], isems.at[slot]
        ).wait()

    def wait_out(slot):
        pltpu.make_async_copy(
            io_v.at[slot], out_hbm.at[pl.ds(0, CE)], osems.at[slot]
        ).wait()

    def wait_emb(slot):
        pltpu.make_async_copy(
            emb_hbm.at[pl.ds(0, CE)], emb_v.at[slot], esems.at[slot]
        ).wait()

    issue_emb(0, 0)
    for b in range(BATCH):
        issue_in(0, b, b)

    def step(k, cur, nxt):
        ecur = cur // 4
        enxt = nxt // 4
        wait_emb(ecur)

        @pl.when(k + 1 < NCHUNK)
        def _():
            issue_emb(k + 1, enxt)

        for b in range(BATCH):
            wait_in(cur + b)

            @pl.when(k >= 1)
            def _():
                wait_out(nxt + b)

            @pl.when(k + 1 < NCHUNK)
            def _():
                issue_in(k + 1, b, nxt + b)

            _add_chunk(io_v.at[cur + b], emb_v.at[ecur])
            issue_out(k, b, cur + b)

    def two_steps(kk, _):
        step(2 * kk, 0, 4)
        step(2 * kk + 1, 4, 0)
        return 0

    lax.fori_loop(0, NCHUNK // 2, two_steps, 0, unroll=False)

    for b in range(BATCH):
        wait_out(4 + b)


def kernel(inputs, embeddings):
    seq_len = inputs.shape[1]
    pos = embeddings[:seq_len].reshape(-1)
    flat = inputs.reshape(-1)
    out = _sc_add(flat, pos)
    return out.reshape(inputs.shape)


# speedup vs baseline: 4.6143x; 3.4332x over previous
import functools

import jax
import jax.numpy as jnp
from jax import lax
from jax.experimental import pallas as pl
from jax.experimental.pallas import tpu as pltpu
from jax.experimental.pallas import tpu_sc as plsc

BATCH = 4
SEQ = 8192
DIM = 1024

NC = 2
NS = 16
NW = NC * NS

ROWS_PER_W = SEQ // NW
CHUNK = 8
CE = CHUNK * DIM
NCHUNK = ROWS_PER_W // CHUNK
UNROLL = 8

_mesh = plsc.VectorSubcoreMesh(
    core_axis_name="c", subcore_axis_name="s", num_cores=NC, num_subcores=NS
)


def _add_chunk(dst, src):
    for r in range(CHUNK):

        @plsc.parallel_loop(0, DIM, 16, unroll=UNROLL)
        def _(off):
            plsc.addupdate(dst.at[r, pl.ds(off, 16)], src[r, pl.ds(off, 16)])


@functools.partial(
    pl.kernel,
    out_type=jax.ShapeDtypeStruct((BATCH, SEQ, DIM), jnp.float32),
    mesh=_mesh,
    scratch_types=[
        pltpu.VMEM((8, CHUNK, DIM), jnp.float32),
        pltpu.VMEM((2, CHUNK, DIM), jnp.float32),
        pltpu.SemaphoreType.DMA((8,)),
        pltpu.SemaphoreType.DMA((8,)),
        pltpu.SemaphoreType.DMA((2,)),
    ],
)
def _sc_add(in_hbm, emb_hbm, out_hbm, io_v, emb_v, isems, osems, esems):
    wid = lax.axis_index("s") * NC + lax.axis_index("c")
    rbase = wid * ROWS_PER_W

    def issue_emb(k, slot):
        pltpu.async_copy(
            emb_hbm.at[pl.ds(rbase + k * CHUNK, CHUNK)],
            emb_v.at[slot],
            esems.at[slot],
        )

    def issue_in(k, b, slot):
        pltpu.async_copy(
            in_hbm.at[b, pl.ds(rbase + k * CHUNK, CHUNK)],
            io_v.at[slot],
            isems.at[slot],
        )

    def issue_out(k, b, slot):
        pltpu.async_copy(
            io_v.at[slot],
            out_hbm.at[b, pl.ds(rbase + k * CHUNK, CHUNK)],
            osems.at[slot],
        )

    def wait_in(slot):
        pltpu.make_async_copy(
            in_hbm.at[0, pl.ds(0, CHUNK)], io_v.at[slot], isems.at[slot]
        ).wait()

    def wait_out(slot):
        pltpu.make_async_copy(
            io_v.at[slot], out_hbm.at[0, pl.ds(0, CHUNK)], osems.at[slot]
        ).wait()

    def wait_emb(slot):
        pltpu.make_async_copy(
            emb_hbm.at[pl.ds(0, CHUNK)], emb_v.at[slot], esems.at[slot]
        ).wait()

    issue_emb(0, 0)
    for b in range(BATCH):
        issue_in(0, b, b)

    def step(k, cur, nxt):
        ecur = cur // 4
        enxt = nxt // 4
        wait_emb(ecur)

        @pl.when(k + 1 < NCHUNK)
        def _():
            issue_emb(k + 1, enxt)

        for b in range(BATCH):
            wait_in(cur + b)

            @pl.when(k >= 1)
            def _():
                wait_out(nxt + b)

            @pl.when(k + 1 < NCHUNK)
            def _():
                issue_in(k + 1, b, nxt + b)

            _add_chunk(io_v.at[cur + b], emb_v.at[ecur])
            issue_out(k, b, cur + b)

    def two_steps(kk, _):
        step(2 * kk, 0, 4)
        step(2 * kk + 1, 4, 0)
        return 0

    lax.fori_loop(0, NCHUNK // 2, two_steps, 0, unroll=False)

    for b in range(BATCH):
        wait_out(4 + b)


def kernel(inputs, embeddings):
    seq_len = inputs.shape[1]
    return _sc_add(inputs, embeddings[:seq_len])
